# baseline (device time: 87692 ns/iter reference)
import jax
import jax.numpy as jnp
from jax import lax
from jax.experimental import pallas as pl
from jax.experimental.pallas import tpu as pltpu


def kernel(ids, E):
    t = ids.shape[0]
    v_local, d = E.shape
    h = t // 2

    my_x = lax.axis_index("x")
    lo = my_x * v_local
    local = ids - lo
    mask = (local >= 0) & (local < v_local)
    idx = jnp.where(mask, local, 0)
    partial = jnp.where(mask[:, None], E[idx], jnp.float32(0))

    def body(p_ref, out_ref, cx_ref, cy_ref, send_sems, recv_sems):
        mx = lax.axis_index("x")
        my = lax.axis_index("y")
        nbr_x = (1 - mx, my)
        nbr_y = (mx, 1 - my)

        barrier = pltpu.get_barrier_semaphore()
        for nbr in (nbr_x, nbr_y):
            pl.semaphore_signal(
                barrier, inc=1, device_id=nbr,
                device_id_type=pl.DeviceIdType.MESH,
            )
        pl.semaphore_wait(barrier, 2)

        my_off = my * h
        other_off = (1 - my) * h

        rdma_x = pltpu.make_async_remote_copy(
            src_ref=p_ref.at[pl.ds(my_off, h), :],
            dst_ref=cx_ref,
            send_sem=send_sems.at[0],
            recv_sem=recv_sems.at[0],
            device_id=nbr_x,
            device_id_type=pl.DeviceIdType.MESH,
        )
        rdma_x.start()
        rdma_x.wait()
        out_ref[pl.ds(my_off, h), :] = p_ref[pl.ds(my_off, h), :] + cx_ref[:, :]

        rdma_y = pltpu.make_async_remote_copy(
            src_ref=out_ref.at[pl.ds(my_off, h), :],
            dst_ref=cy_ref,
            send_sem=send_sems.at[1],
            recv_sem=recv_sems.at[1],
            device_id=nbr_y,
            device_id_type=pl.DeviceIdType.MESH,
        )
        rdma_y.start()
        rdma_y.wait()
        out_ref[pl.ds(other_off, h), :] = cy_ref[:, :]

    return pl.pallas_call(
        body,
        out_shape=jax.ShapeDtypeStruct((t, d), jnp.float32),
        in_specs=[pl.BlockSpec(memory_space=pltpu.VMEM)],
        out_specs=pl.BlockSpec(memory_space=pltpu.VMEM),
        scratch_shapes=[
            pltpu.VMEM((h, d), jnp.float32),
            pltpu.VMEM((h, d), jnp.float32),
            pltpu.SemaphoreType.DMA((2,)),
            pltpu.SemaphoreType.DMA((2,)),
        ],
        compiler_params=pltpu.CompilerParams(collective_id=0),
    )(partial)


# device time: 47251 ns/iter; 1.8559x vs baseline; 1.8559x over previous
import jax
import jax.numpy as jnp
from jax import lax
from jax.experimental import pallas as pl
from jax.experimental.pallas import tpu as pltpu

NCHUNK = 8


def kernel(ids, E):
    t = ids.shape[0]
    v_local, d = E.shape
    h = t // 2
    ch = h // NCHUNK

    my_x = lax.axis_index("x")
    my_y = lax.axis_index("y")
    lo = my_x * v_local

    ids_half = lax.dynamic_slice(ids, (my_y * h,), (h,))
    local = ids_half - lo
    mask = (local >= 0) & (local < v_local)
    idx = jnp.where(mask, local, 0)
    partial = jnp.where(mask[:, None], E[idx], jnp.float32(0))

    def body(p_ref, out_ref, cx_ref, cy_ref, sx, rx, sy, ry):
        mx = lax.axis_index("x")
        my = lax.axis_index("y")
        nbr_x = (1 - mx, my)
        nbr_y = (mx, 1 - my)

        barrier = pltpu.get_barrier_semaphore()
        for nbr in (nbr_x, nbr_y):
            pl.semaphore_signal(
                barrier, inc=1, device_id=nbr,
                device_id_type=pl.DeviceIdType.MESH,
            )
        pl.semaphore_wait(barrier, 2)

        my_off = my * h
        other_off = (1 - my) * h

        x_rdmas = []
        for i in range(NCHUNK):
            r = pltpu.make_async_remote_copy(
                src_ref=p_ref.at[pl.ds(i * ch, ch), :],
                dst_ref=cx_ref.at[pl.ds(i * ch, ch), :],
                send_sem=sx.at[i],
                recv_sem=rx.at[i],
                device_id=nbr_x,
                device_id_type=pl.DeviceIdType.MESH,
            )
            r.start()
            x_rdmas.append(r)

        y_rdmas = []
        for i in range(NCHUNK):
            x_rdmas[i].wait_recv()
            out_ref[pl.ds(my_off + i * ch, ch), :] = (
                p_ref[pl.ds(i * ch, ch), :] + cx_ref[pl.ds(i * ch, ch), :]
            )
            r = pltpu.make_async_remote_copy(
                src_ref=out_ref.at[pl.ds(my_off + i * ch, ch), :],
                dst_ref=cy_ref.at[pl.ds(i * ch, ch), :],
                send_sem=sy.at[i],
                recv_sem=ry.at[i],
                device_id=nbr_y,
                device_id_type=pl.DeviceIdType.MESH,
            )
            r.start()
            y_rdmas.append(r)

        for i in range(NCHUNK):
            y_rdmas[i].wait_recv()
            out_ref[pl.ds(other_off + i * ch, ch), :] = cy_ref[
                pl.ds(i * ch, ch), :
            ]

        for i in range(NCHUNK):
            x_rdmas[i].wait_send()
            y_rdmas[i].wait_send()

    return pl.pallas_call(
        body,
        out_shape=jax.ShapeDtypeStruct((t, d), jnp.float32),
        in_specs=[pl.BlockSpec(memory_space=pltpu.VMEM)],
        out_specs=pl.BlockSpec(memory_space=pltpu.VMEM),
        scratch_shapes=[
            pltpu.VMEM((h, d), jnp.float32),
            pltpu.VMEM((h, d), jnp.float32),
            pltpu.SemaphoreType.DMA((NCHUNK,)),
            pltpu.SemaphoreType.DMA((NCHUNK,)),
            pltpu.SemaphoreType.DMA((NCHUNK,)),
            pltpu.SemaphoreType.DMA((NCHUNK,)),
        ],
        compiler_params=pltpu.CompilerParams(collective_id=0),
    )(partial)
